# Initial kernel scaffold; baseline (speedup 1.0000x reference)
import jax
import jax.numpy as jnp
from jax.experimental import pallas as pl


def kernel(token_seq, hour_seq, poi_table, hour_table, fc_w, fc_b):
    raise NotImplementedError("write your pallas kernel here")


# speedup vs baseline: 1.8949x; 1.8949x over previous
import functools
import jax
import jax.numpy as jnp
from jax import lax
from jax.experimental import pallas as pl
from jax.experimental.pallas import tpu as pltpu
from jax.experimental.pallas import tpu_sc as plsc

B, L = 4096, 200
E = 64
FAN_H = 16
NUM_HOURS = 25
HN = 32
N_TOK = B * L
P = 1000000 + 1
NW = 32
PER_W = N_TOK // NW
CHUNK = 512
N_CHUNKS = PER_W // CHUNK
NBUF = 2
T = 2048
G = N_TOK // T


@functools.lru_cache(maxsize=None)
def _make_sc_gather():
    mesh = plsc.VectorSubcoreMesh(core_axis_name="c", subcore_axis_name="s")

    @functools.partial(
        pl.kernel,
        mesh=mesh,
        out_type=jax.ShapeDtypeStruct((N_TOK, E), jnp.float32),
        scratch_types=[
            pltpu.VMEM((NBUF, CHUNK), jnp.int32),
            pltpu.VMEM((NBUF, CHUNK, E), jnp.float32),
            pltpu.SemaphoreType.DMA,
        ],
        compiler_params=pltpu.CompilerParams(use_tc_tiling_on_sc=False),
    )
    def _sc_gather(idx_hbm, table_hbm, out_hbm, idx_v, rows_v, sem):
        wid = lax.axis_index("s") * 2 + lax.axis_index("c")
        base = wid * PER_W

        def body(i, carry):
            slot = lax.rem(i, NBUF)
            off = pl.multiple_of(base + i * CHUNK, CHUNK)
            pltpu.sync_copy(idx_hbm.at[pl.ds(off, CHUNK)], idx_v.at[slot])
            pltpu.async_copy(table_hbm.at[idx_v.at[slot]], rows_v.at[slot],
                             sem).wait()
            pltpu.sync_copy(rows_v.at[slot], out_hbm.at[pl.ds(off, CHUNK)])
            return carry

        lax.fori_loop(0, N_CHUNKS, body, 0)

    return _sc_gather


def _dense_body(hour_ref, x_ref, fcw_ref, hpad_ref, b_ref, out_ref):
    x = x_ref[...]
    w1 = fcw_ref[:, :E]
    acc = lax.dot_general(x, w1, (((1,), (1,)), ((), ())),
                          preferred_element_type=jnp.float32)
    w2 = fcw_ref[:, E:]
    hw = lax.dot_general(hpad_ref[...], w2, (((1,), (1,)), ((), ())),
                         preferred_element_type=jnp.float32)
    hw = hw + b_ref[...]
    oh = (lax.broadcasted_iota(jnp.int32, (HN, T), 0)
          == hour_ref[0]).astype(jnp.float32)
    hc = lax.dot_general(oh, hw, (((0,), (0,)), ((), ())),
                         preferred_element_type=jnp.float32)
    out_ref[...] = jnp.tanh(acc + hc)


def _dense(hour3, x, fc_w, hour_pad, fc_b2):
    return pl.pallas_call(
        _dense_body,
        grid=(G,),
        in_specs=[
            pl.BlockSpec((1, 1, T), lambda i: (i, 0, 0)),
            pl.BlockSpec((T, E), lambda i: (i, 0)),
            pl.BlockSpec((E, E + FAN_H), lambda i: (0, 0)),
            pl.BlockSpec((HN, FAN_H), lambda i: (0, 0)),
            pl.BlockSpec((1, E), lambda i: (0, 0)),
        ],
        out_specs=pl.BlockSpec((T, E), lambda i: (i, 0)),
        out_shape=jax.ShapeDtypeStruct((N_TOK, E), jnp.float32),
    )(hour3, x, fc_w, hour_pad, fc_b2)


def kernel(token_seq, hour_seq, poi_table, hour_table, fc_w, fc_b):
    tok = jnp.asarray(token_seq, jnp.int32).reshape(N_TOK)
    hour3 = jnp.asarray(hour_seq, jnp.int32).reshape(G, 1, T)
    hour_pad = jnp.pad(hour_table.astype(jnp.float32),
                       ((0, HN - NUM_HOURS), (0, 0)))
    fc_b2 = fc_b.astype(jnp.float32).reshape(1, E)
    x = _make_sc_gather()(tok, poi_table.astype(jnp.float32))
    out = _dense(hour3, x, fc_w.astype(jnp.float32), hour_pad, fc_b2)
    return out.reshape(B, L, E)


# speedup vs baseline: 2.1393x; 1.1290x over previous
import functools
import jax
import jax.numpy as jnp
from jax import lax
from jax.experimental import pallas as pl
from jax.experimental.pallas import tpu as pltpu
from jax.experimental.pallas import tpu_sc as plsc

B, L = 4096, 200
E = 64
FAN_H = 16
NUM_HOURS = 25
HN = 32
N_TOK = B * L
P = 1000000 + 1
NW = 32
PER_W = N_TOK // NW
CHUNK = 512
N_CHUNKS = PER_W // CHUNK
NBUF = 2
T = 2048
G = N_TOK // T


@functools.lru_cache(maxsize=None)
def _make_sc_gather():
    mesh = plsc.VectorSubcoreMesh(core_axis_name="c", subcore_axis_name="s")

    @functools.partial(
        pl.kernel,
        mesh=mesh,
        out_type=jax.ShapeDtypeStruct((N_TOK, E), jnp.float32),
        scratch_types=[
            pltpu.VMEM((NBUF, CHUNK), jnp.int32),
            pltpu.VMEM((NBUF, CHUNK, E), jnp.float32),
            pltpu.SemaphoreType.DMA,
        ],
        compiler_params=pltpu.CompilerParams(use_tc_tiling_on_sc=False),
    )
    def _sc_gather(idx_hbm, table_hbm, out_hbm, idx_v, rows_v, sem):
        wid = lax.axis_index("s") * 2 + lax.axis_index("c")
        base = wid * PER_W

        def body(i, carry):
            slot = lax.rem(i, NBUF)
            off = pl.multiple_of(base + i * CHUNK, CHUNK)
            pltpu.sync_copy(idx_hbm.at[pl.ds(off, CHUNK)], idx_v.at[slot])
            pltpu.async_copy(table_hbm.at[idx_v.at[slot]], rows_v.at[slot],
                             sem).wait()
            pltpu.sync_copy(rows_v.at[slot], out_hbm.at[pl.ds(off, CHUNK)])
            return carry

        lax.fori_loop(0, N_CHUNKS, body, 0)

    return _sc_gather


def _dense_body(he_ref, ho_ref, x2_ref, fcw_ref, hpad_ref, b_ref, out_ref):
    x2 = x2_ref[...]
    w1 = fcw_ref[:, :E]
    acce = lax.dot_general(x2[:, :E], w1, (((1,), (1,)), ((), ())),
                           preferred_element_type=jnp.float32)
    acco = lax.dot_general(x2[:, E:], w1, (((1,), (1,)), ((), ())),
                           preferred_element_type=jnp.float32)
    w2 = fcw_ref[:, E:]
    hw = lax.dot_general(hpad_ref[...], w2, (((1,), (1,)), ((), ())),
                         preferred_element_type=jnp.float32)
    hw = hw + b_ref[...]
    ioh = lax.broadcasted_iota(jnp.int32, (HN, T // 2), 0)
    ohe = (ioh == he_ref[0]).astype(jnp.float32)
    oho = (ioh == ho_ref[0]).astype(jnp.float32)
    hce = lax.dot_general(ohe, hw, (((0,), (0,)), ((), ())),
                          preferred_element_type=jnp.float32)
    hco = lax.dot_general(oho, hw, (((0,), (0,)), ((), ())),
                          preferred_element_type=jnp.float32)
    ye = jnp.tanh(acce + hce)
    yo = jnp.tanh(acco + hco)
    out_ref[...] = jnp.stack([ye, yo], axis=1).reshape(T, E)


def _dense(he3, ho3, x2, fc_w, hour_pad, fc_b2):
    return pl.pallas_call(
        _dense_body,
        grid=(G,),
        in_specs=[
            pl.BlockSpec((1, 1, T // 2), lambda i: (i, 0, 0)),
            pl.BlockSpec((1, 1, T // 2), lambda i: (i, 0, 0)),
            pl.BlockSpec((T // 2, 2 * E), lambda i: (i, 0)),
            pl.BlockSpec((E, E + FAN_H), lambda i: (0, 0)),
            pl.BlockSpec((HN, FAN_H), lambda i: (0, 0)),
            pl.BlockSpec((1, E), lambda i: (0, 0)),
        ],
        out_specs=pl.BlockSpec((T, E), lambda i: (i, 0)),
        out_shape=jax.ShapeDtypeStruct((N_TOK, E), jnp.float32),
    )(he3, ho3, x2, fc_w, hour_pad, fc_b2)


def kernel(token_seq, hour_seq, poi_table, hour_table, fc_w, fc_b):
    tok = jnp.asarray(token_seq, jnp.int32).reshape(N_TOK)
    hour = jnp.asarray(hour_seq, jnp.int32).reshape(N_TOK)
    he3 = hour[0::2].reshape(G, 1, T // 2)
    ho3 = hour[1::2].reshape(G, 1, T // 2)
    hour_pad = jnp.pad(hour_table.astype(jnp.float32),
                       ((0, HN - NUM_HOURS), (0, 0)))
    fc_b2 = fc_b.astype(jnp.float32).reshape(1, E)
    x = _make_sc_gather()(tok, poi_table.astype(jnp.float32))
    x2 = x.reshape(N_TOK // 2, 2 * E)
    out = _dense(he3, ho3, x2, fc_w.astype(jnp.float32), hour_pad, fc_b2)
    return out.reshape(B, L, E)
